# Initial kernel scaffold; baseline (speedup 1.0000x reference)
import jax
import jax.numpy as jnp
from jax.experimental import pallas as pl


def kernel(audio_codes, embed_table):
    raise NotImplementedError("write your pallas kernel here")


# speedup vs baseline: 1.5569x; 1.5569x over previous
import functools

import jax
import jax.numpy as jnp
from jax import lax
from jax.experimental import pallas as pl
from jax.experimental.pallas import tpu as pltpu
from jax.experimental.pallas import tpu_sc as plsc

VOCAB = 1028
C = 9
H = 2048
NC = 2
NS = 16
L = 16
NW = NC * NS

HALVES = 2
HH = H // HALVES
P = 8


def _build_sc_kernel(n_pos: int):
    per_w = n_pos // NW
    n_groups = per_w // P

    mesh = plsc.VectorSubcoreMesh(core_axis_name="c", subcore_axis_name="s")

    @functools.partial(
        pl.kernel,
        mesh=mesh,
        out_type=jax.ShapeDtypeStruct((n_pos, H), jnp.float32),
        scratch_types=[
            pltpu.VMEM((HALVES * C * per_w,), jnp.int32),
            pltpu.VMEM((C * P, HH), jnp.float32),
            pltpu.VMEM((P, HH), jnp.float32),
        ],
    )
    def k(idx_hbm, table_hbm, out_hbm, idx_v, rows_v, stage_v):
        wid = lax.axis_index("s") * NC + lax.axis_index("c")
        base = wid * per_w
        for h in range(HALVES):
            for c in range(C):
                pltpu.sync_copy(
                    idx_hbm.at[pl.ds((h * C + c) * n_pos + base, per_w)],
                    idx_v.at[pl.ds((h * C + c) * per_w, per_w)])

        @pl.loop(0, n_groups)
        def _(g):
            p0 = g * P
            for h in range(HALVES):
                for c in range(C):
                    pltpu.sync_copy(
                        table_hbm.at[
                            idx_v.at[pl.ds((h * C + c) * per_w + p0, P)]],
                        rows_v.at[pl.ds(c * P, P)])
                for r in range(P):
                    @pl.loop(0, HH, step=L)
                    def _(j):
                        v = rows_v[r, pl.ds(j, L)]
                        for c in range(1, C):
                            v = v + rows_v[c * P + r, pl.ds(j, L)]
                        stage_v[r, pl.ds(j, L)] = v
                pltpu.sync_copy(
                    stage_v,
                    out_hbm.at[pl.ds(base + p0, P), pl.ds(h * HH, HH)])

    return k


def kernel(audio_codes, embed_table):
    b, s, c = audio_codes.shape
    n_pos = b * s
    offs = jnp.arange(C, dtype=jnp.int32) * VOCAB
    tok = (audio_codes.astype(jnp.int32).reshape(n_pos, C) + offs).T
    idx = jnp.stack([2 * tok, 2 * tok + 1]).reshape(-1)
    table2 = embed_table.reshape(-1, HH)
    out = _build_sc_kernel(n_pos)(idx, table2)
    return out.reshape(b, s, H)


# speedup vs baseline: 2.1213x; 1.3625x over previous
import functools

import jax
import jax.numpy as jnp
from jax import lax
from jax.experimental import pallas as pl
from jax.experimental.pallas import tpu as pltpu
from jax.experimental.pallas import tpu_sc as plsc

VOCAB = 1028
C = 9
H = 2048
NC = 2
NS = 16
L = 16
NW = NC * NS

Q = 4
W = H // Q
P = 8
GROUP = C * P


def _build_sc_kernel(n_pos: int):
    per_w = n_pos // NW
    g_per_w = per_w // P
    items = Q * g_per_w
    n_groups = n_pos // P

    mesh = plsc.VectorSubcoreMesh(core_axis_name="c", subcore_axis_name="s")

    @functools.partial(
        pl.kernel,
        mesh=mesh,
        out_type=jax.ShapeDtypeStruct((n_pos, H), jnp.float32),
        scratch_types=[
            pltpu.VMEM((items * GROUP,), jnp.int32),
            pltpu.VMEM((GROUP, W), jnp.float32),
            pltpu.VMEM((GROUP, W), jnp.float32),
            pltpu.VMEM((P, W), jnp.float32),
            pltpu.VMEM((P, W), jnp.float32),
            pltpu.SemaphoreType.DMA,
            pltpu.SemaphoreType.DMA,
            pltpu.SemaphoreType.DMA,
            pltpu.SemaphoreType.DMA,
        ],
    )
    def k(idx_hbm, table_hbm, out_hbm, idx_v, rows0, rows1, stage0, stage1,
          gsem0, gsem1, ssem0, ssem1):
        wid = lax.axis_index("s") * NC + lax.axis_index("c")
        base = wid * per_w
        blk = g_per_w * GROUP
        for q in range(Q):
            pltpu.sync_copy(
                idx_hbm.at[pl.ds((q * n_groups + wid * g_per_w) * GROUP, blk)],
                idx_v.at[pl.ds(q * blk, blk)])

        def fire_gather(t, rows, sem):
            pltpu.async_copy(
                table_hbm.at[idx_v.at[pl.ds(t * GROUP, GROUP)]], rows, sem)

        def wait_gather(rows, sem):
            pltpu.make_async_copy(
                table_hbm.at[idx_v.at[pl.ds(0, GROUP)]], rows, sem).wait()

        def out_slice(t):
            q = lax.shift_right_logical(t, 4)
            g = lax.bitwise_and(t, g_per_w - 1)
            return out_hbm.at[pl.ds(base + g * P, P), pl.ds(q * W, W)]

        def fire_store(t, stage, sem):
            pltpu.async_copy(stage, out_slice(t), sem)

        def wait_store(stage, sem):
            pltpu.make_async_copy(
                stage, out_hbm.at[pl.ds(0, P), pl.ds(0, W)], sem).wait()

        def compute(rows, stage):
            for r in range(P):
                @pl.loop(0, W, step=2 * L)
                def _(j):
                    for jj in (0, L):
                        v = rows[r, pl.ds(j + jj, L)]
                        for c in range(1, C):
                            v = v + rows[c * P + r, pl.ds(j + jj, L)]
                        stage[r, pl.ds(j + jj, L)] = v

        fire_gather(0, rows0, gsem0)

        @pl.loop(0, items // 2)
        def _(k2):
            t0 = 2 * k2
            wait_gather(rows0, gsem0)
            fire_gather(t0 + 1, rows1, gsem1)

            @pl.when(k2 > 0)
            def _():
                wait_store(stage0, ssem0)
            compute(rows0, stage0)
            fire_store(t0, stage0, ssem0)
            @pl.when(k2 < items // 2 - 1)
            def _():
                fire_gather(t0 + 2, rows0, gsem0)
            wait_gather(rows1, gsem1)

            @pl.when(k2 > 0)
            def _():
                wait_store(stage1, ssem1)
            compute(rows1, stage1)
            fire_store(t0 + 1, stage1, ssem1)

        wait_store(stage0, ssem0)
        wait_store(stage1, ssem1)

    return k


def kernel(audio_codes, embed_table):
    b, s, _ = audio_codes.shape
    n_pos = b * s
    n_groups = n_pos // P
    offs = jnp.arange(C, dtype=jnp.int32) * VOCAB
    tok = (audio_codes.astype(jnp.int32).reshape(n_pos, C) + offs).T
    t = tok.reshape(C, n_groups, P).transpose(1, 0, 2)
    idx = (Q * t)[None] + jnp.arange(Q, dtype=jnp.int32).reshape(Q, 1, 1, 1)
    idx = idx.reshape(-1)
    table4 = embed_table.reshape(-1, W)
    out = _build_sc_kernel(n_pos)(idx, table4)
    return out.reshape(b, s, H)


# speedup vs baseline: 2.2900x; 1.0796x over previous
import functools

import jax
import jax.numpy as jnp
from jax import lax
from jax.experimental import pallas as pl
from jax.experimental.pallas import tpu as pltpu
from jax.experimental.pallas import tpu_sc as plsc

VOCAB = 1028
C = 9
H = 2048
NC = 2
NS = 16
L = 16
NW = NC * NS

Q = 4
W = H // Q
P = 8
GROUP = C * P


def _build_sc_kernel(n_pos: int):
    per_w = n_pos // NW
    g_per_w = per_w // P
    items = Q * g_per_w
    n_groups = n_pos // P

    mesh = plsc.VectorSubcoreMesh(core_axis_name="c", subcore_axis_name="s")

    @functools.partial(
        pl.kernel,
        mesh=mesh,
        out_type=jax.ShapeDtypeStruct((n_pos, H), jnp.float32),
        scratch_types=[
            pltpu.VMEM((items * GROUP,), jnp.int32),
            pltpu.VMEM((GROUP, W), jnp.float32),
            pltpu.VMEM((GROUP, W), jnp.float32),
            pltpu.VMEM((P, W), jnp.float32),
            pltpu.VMEM((P, W), jnp.float32),
            pltpu.SemaphoreType.DMA,
            pltpu.SemaphoreType.DMA,
            pltpu.SemaphoreType.DMA,
            pltpu.SemaphoreType.DMA,
        ],
    )
    def k(idx_hbm, table_hbm, out_hbm, idx_v, rows0, rows1, stage0, stage1,
          gsem0, gsem1, ssem0, ssem1):
        wid = lax.axis_index("s") * NC + lax.axis_index("c")
        base = wid * per_w
        blk = g_per_w * GROUP
        for q in range(Q):
            pltpu.sync_copy(
                idx_hbm.at[pl.ds((q * n_groups + wid * g_per_w) * GROUP, blk)],
                idx_v.at[pl.ds(q * blk, blk)])

        def fire_gather(t, rows, sem):
            pltpu.async_copy(
                table_hbm.at[idx_v.at[pl.ds(t * GROUP, GROUP)]], rows, sem)

        def wait_gather(rows, sem):
            pltpu.make_async_copy(
                table_hbm.at[idx_v.at[pl.ds(0, GROUP)]], rows, sem).wait()

        def out_slice(t):
            q = lax.shift_right_logical(t, 4)
            g = lax.bitwise_and(t, g_per_w - 1)
            return out_hbm.at[pl.ds(base + g * P, P), pl.ds(q * W, W)]

        def fire_store(t, stage, sem):
            pltpu.async_copy(stage, out_slice(t), sem)

        def wait_store(stage, sem):
            pltpu.make_async_copy(
                stage, out_hbm.at[pl.ds(0, P), pl.ds(0, W)], sem).wait()

        def compute(rows, stage):
            for r in range(P):
                @pl.loop(0, W, step=4 * L)
                def _(j):
                    for jj in range(0, 4 * L, L):
                        sl = pl.ds(j + jj, L)
                        vs = [rows[r * C + c, sl] for c in range(C)]
                        while len(vs) > 1:
                            nxt = [vs[i] + vs[i + 1]
                                   for i in range(0, len(vs) - 1, 2)]
                            if len(vs) % 2:
                                nxt.append(vs[-1])
                            vs = nxt
                        stage[r, sl] = vs[0]

        fire_gather(0, rows0, gsem0)

        @pl.loop(0, items // 2)
        def _(k2):
            t0 = 2 * k2
            wait_gather(rows0, gsem0)
            fire_gather(t0 + 1, rows1, gsem1)

            @pl.when(k2 > 0)
            def _():
                wait_store(stage0, ssem0)
            compute(rows0, stage0)
            fire_store(t0, stage0, ssem0)
            @pl.when(k2 < items // 2 - 1)
            def _():
                fire_gather(t0 + 2, rows0, gsem0)
            wait_gather(rows1, gsem1)

            @pl.when(k2 > 0)
            def _():
                wait_store(stage1, ssem1)
            compute(rows1, stage1)
            fire_store(t0 + 1, stage1, ssem1)

        wait_store(stage0, ssem0)
        wait_store(stage1, ssem1)

    return k


def kernel(audio_codes, embed_table):
    b, s, _ = audio_codes.shape
    n_pos = b * s
    n_groups = n_pos // P
    offs = jnp.arange(C, dtype=jnp.int32) * VOCAB
    tok = audio_codes.astype(jnp.int32).reshape(n_pos, C) + offs
    idx = (Q * tok)[None] + jnp.arange(Q, dtype=jnp.int32).reshape(Q, 1, 1)
    idx = idx.reshape(-1)
    table4 = embed_table.reshape(-1, W)
    out = _build_sc_kernel(n_pos)(idx, table4)
    return out.reshape(b, s, H)
